# baseline (speedup 1.0000x reference)
import jax
import jax.numpy as jnp
from jax import lax
from jax.experimental import pallas as pl
from jax.experimental.pallas import tpu as pltpu
from jax.experimental.pallas import tpu_sc as plsc

NC = 2
NS = 16
L = 16
NW = NC * NS
BATCH = 16384
D = 32
PACK = 128 // D
BPW = BATCH // NW
CHUNK = 128
NCHUNK = BPW // CHUNK
NPASS = 2
RPP = BPW // NPASS
CPP = RPP // CHUNK
GPP = RPP // L


def _sc_body(uids_hbm, aids_hbm, utab_hbm, atab_hbm, out_hbm,
             uraw_v, araw_v, uwid_v, awid_v, uwide_v, awide_v, out_v, sem):
    wid = lax.axis_index("s") * NC + lax.axis_index("c")
    base = wid * BPW

    pltpu.sync_copy(uids_hbm.at[pl.ds(base, BPW)], uraw_v)
    pltpu.sync_copy(aids_hbm.at[pl.ds(base, BPW)], araw_v)

    for j in range(NCHUNK):
        for k in range(CHUNK // L):
            s = pl.ds(k * L, L)
            uwid_v[j, s] = lax.shift_right_logical(
                uraw_v[pl.ds(j * CHUNK + k * L, L)], 2)
            awid_v[j, s] = lax.shift_right_logical(
                araw_v[pl.ds(j * CHUNK + k * L, L)], 2)

    iota = lax.iota(jnp.int32, L)

    for p in range(NPASS):
        copies = []
        for c in range(CPP):
            j = p * CPP + c
            copies.append(pltpu.make_async_copy(
                utab_hbm.at[uwid_v.at[j]],
                uwide_v.at[pl.ds(c * CHUNK, CHUNK)], sem))
            copies.append(pltpu.make_async_copy(
                atab_hbm.at[awid_v.at[j]],
                awide_v.at[pl.ds(c * CHUNK, CHUNK)], sem))
        for cp in copies:
            cp.start()
        for cp in copies:
            cp.wait()

        @pl.loop(0, GPP)
        def _group(gg):
            goff = p * RPP + gg * L
            uid = uraw_v[pl.ds(goff, L)]
            aid = araw_v[pl.ds(goff, L)]
            rid = gg * L + iota
            ucol = lax.shift_left(jnp.bitwise_and(uid, 3), 5)
            acol = lax.shift_left(jnp.bitwise_and(aid, 3), 5)
            acc = jnp.zeros((L,), jnp.float32)
            for d in range(D):
                u = plsc.load_gather(uwide_v, [rid, ucol + d])
                a = plsc.load_gather(awide_v, [rid, acol + d])
                acc = acc + u * a
            ps = 1.0 / (1.0 + jnp.exp(-acc))
            oid = (goff + iota) * 2
            plsc.store_scatter(out_v, [oid], 1.0 - ps)
            plsc.store_scatter(out_v, [oid + 1], ps)

    pltpu.sync_copy(out_v, out_hbm.at[pl.ds(base * 2, BPW * 2)])


def kernel(user_ids, ad_ids, user_table, ad_table):
    uids = user_ids.astype(jnp.int32)
    aids = ad_ids.astype(jnp.int32)
    utab = user_table.reshape(user_table.shape[0] // PACK, 128)
    atab = ad_table.reshape(ad_table.shape[0] // PACK, 128)
    mesh = plsc.VectorSubcoreMesh(core_axis_name="c", subcore_axis_name="s",
                                  num_cores=NC, num_subcores=NS)
    f = pl.kernel(
        _sc_body,
        out_type=jax.ShapeDtypeStruct((BATCH * 2,), jnp.float32),
        mesh=mesh,
        compiler_params=pltpu.CompilerParams(needs_layout_passes=False),
        scratch_types=[
            pltpu.VMEM((BPW,), jnp.int32),
            pltpu.VMEM((BPW,), jnp.int32),
            pltpu.VMEM((NCHUNK, CHUNK), jnp.int32),
            pltpu.VMEM((NCHUNK, CHUNK), jnp.int32),
            pltpu.VMEM((RPP, 128), jnp.float32),
            pltpu.VMEM((RPP, 128), jnp.float32),
            pltpu.VMEM((BPW * 2,), jnp.float32),
            pltpu.SemaphoreType.DMA,
        ],
    )
    out = f(uids, aids, utab, atab)
    return out.reshape(BATCH, 2)


# speedup vs baseline: 2.6124x; 2.6124x over previous
import jax
import jax.numpy as jnp
from jax import lax
from jax.experimental import pallas as pl
from jax.experimental.pallas import tpu as pltpu
from jax.experimental.pallas import tpu_sc as plsc

NC = 2
NS = 16
L = 16
NW = NC * NS
BATCH = 16384
D = 32
BPW = BATCH // NW
K = 4


def _sc_body(uids_hbm, aids_hbm, utab_hbm, atab_hbm, out_hbm,
             uids_v, aids_v, ubufs, abufs, dots_v, out_v, sems):
    wid = lax.axis_index("s") * NC + lax.axis_index("c")
    base = wid * BPW

    pltpu.sync_copy(uids_hbm.at[pl.ds(base, BPW)], uids_v.at[pl.ds(0, BPW)])
    pltpu.sync_copy(aids_hbm.at[pl.ds(base, BPW)], aids_v.at[pl.ds(0, BPW)])

    def _sread(ref, e):
        return ref[pl.ds(e, L)][0]

    iota = lax.iota(jnp.int32, L)
    lo_rows = iota
    hi_rows = iota + L

    def _issue(e, k):
        uid = _sread(uids_v, e)
        aid = _sread(aids_v, e)
        ub = pl.multiple_of(lax.shift_left(lax.shift_right_logical(uid, 7), 7),
                            128)
        ar = pl.multiple_of(lax.shift_left(lax.shift_right_logical(aid, 3), 3),
                            8)
        pltpu.async_copy(utab_hbm.at[:, pl.ds(ub, 128)], ubufs.at[k], sems.at[k])
        pltpu.async_copy(atab_hbm.at[pl.ds(ar, 8), :], abufs.at[k], sems.at[k])

    def _drain(k):
        pltpu.make_async_copy(
            utab_hbm.at[:, pl.ds(0, 128)], ubufs.at[k], sems.at[k]).wait()
        pltpu.make_async_copy(
            atab_hbm.at[pl.ds(0, 8), :], abufs.at[k], sems.at[k]).wait()

    def _compute(e, k):
        uid = _sread(uids_v, e)
        aid = _sread(aids_v, e)
        uc = jnp.full((L,), jnp.bitwise_and(uid, 127), jnp.int32)
        arow = jnp.bitwise_and(aid, 7)
        u_lo = plsc.load_gather(ubufs.at[k], [lo_rows, uc])
        u_hi = plsc.load_gather(ubufs.at[k], [hi_rows, uc])
        a_lo = abufs.at[k][arow, pl.ds(0, L)]
        a_hi = abufs.at[k][arow, pl.ds(L, L)]
        prod = u_lo * a_lo + u_hi * a_hi
        return lax.reduce_sum(prod, (0,))

    for k in range(K):
        _issue(k, k)

    zeros = jnp.zeros((L,), jnp.float32)

    @pl.loop(0, BPW // K, init_carry=zeros)
    def _eiter(i, acc):
        e0 = i * K
        for k in range(K):
            _drain(k)
            s = _compute(e0 + k, k)
            lane = jnp.bitwise_and(e0 + k, L - 1)
            acc = jnp.where(iota == lane, s, acc)

            @pl.when(i < BPW // K - 1)
            def _():
                _issue(e0 + k + K, k)

        @pl.when(jnp.bitwise_and(i, (L // K) - 1) == (L // K) - 1)
        def _():
            ps = 1.0 / (1.0 + jnp.exp(-acc))
            g = lax.div(i, L // K)
            oid = (g * L + iota) * 2
            plsc.store_scatter(out_v, [oid], 1.0 - ps)
            plsc.store_scatter(out_v, [oid + 1], ps)

        return acc

    pltpu.sync_copy(out_v, out_hbm.at[pl.ds(base * 2, BPW * 2)])


def kernel(user_ids, ad_ids, user_table, ad_table):
    uids = user_ids.astype(jnp.int32)
    aids = ad_ids.astype(jnp.int32)
    utab = user_table.T
    mesh = plsc.VectorSubcoreMesh(core_axis_name="c", subcore_axis_name="s",
                                  num_cores=NC, num_subcores=NS)
    f = pl.kernel(
        _sc_body,
        out_type=jax.ShapeDtypeStruct((BATCH * 2,), jnp.float32),
        mesh=mesh,
        compiler_params=pltpu.CompilerParams(needs_layout_passes=False),
        scratch_types=[
            pltpu.VMEM((BPW + L,), jnp.int32),
            pltpu.VMEM((BPW + L,), jnp.int32),
            pltpu.VMEM((K, D, 128), jnp.float32),
            pltpu.VMEM((K, 8, D), jnp.float32),
            pltpu.VMEM((BPW,), jnp.float32),
            pltpu.VMEM((BPW * 2,), jnp.float32),
            pltpu.SemaphoreType.DMA((K,)),
        ],
    )
    out = f(uids, aids, utab, ad_table)
    return out.reshape(BATCH, 2)


# speedup vs baseline: 3.0623x; 1.1722x over previous
import jax
import jax.numpy as jnp
from jax import lax
from jax.experimental import pallas as pl
from jax.experimental.pallas import tpu as pltpu
from jax.experimental.pallas import tpu_sc as plsc

NC = 2
NS = 16
L = 16
NW = NC * NS
BATCH = 16384
D = 32
BPW = BATCH // NW
K = 8


def _sc_body(uids_hbm, aids_hbm, utab_hbm, atab_hbm, out_hbm,
             uids_v, aids_v, ubufs, abufs, dots_v, out_v, sems):
    wid = lax.axis_index("s") * NC + lax.axis_index("c")
    base = wid * BPW

    pltpu.sync_copy(uids_hbm.at[pl.ds(base, BPW)], uids_v.at[pl.ds(0, BPW)])
    pltpu.sync_copy(aids_hbm.at[pl.ds(base, BPW)], aids_v.at[pl.ds(0, BPW)])

    def _sread(ref, e):
        return ref[pl.ds(e, L)][0]

    iota = lax.iota(jnp.int32, L)
    lo_rows = iota
    hi_rows = iota + L

    def _issue(e, k):
        uid = _sread(uids_v, e)
        aid = _sread(aids_v, e)
        ub = pl.multiple_of(lax.shift_left(lax.shift_right_logical(uid, 7), 7),
                            128)
        ar = pl.multiple_of(lax.shift_left(lax.shift_right_logical(aid, 3), 3),
                            8)
        pltpu.async_copy(utab_hbm.at[:, pl.ds(ub, 128)], ubufs.at[k], sems.at[k])
        pltpu.async_copy(atab_hbm.at[pl.ds(ar, 8), :], abufs.at[k], sems.at[k])

    def _drain(k):
        pltpu.make_async_copy(
            utab_hbm.at[:, pl.ds(0, 128)], ubufs.at[k], sems.at[k]).wait()
        pltpu.make_async_copy(
            atab_hbm.at[pl.ds(0, 8), :], abufs.at[k], sems.at[k]).wait()

    def _compute(e, k):
        uid = _sread(uids_v, e)
        aid = _sread(aids_v, e)
        uc = jnp.full((L,), jnp.bitwise_and(uid, 127), jnp.int32)
        arow = jnp.bitwise_and(aid, 7)
        u_lo = plsc.load_gather(ubufs.at[k], [lo_rows, uc])
        u_hi = plsc.load_gather(ubufs.at[k], [hi_rows, uc])
        a_lo = abufs.at[k][arow, pl.ds(0, L)]
        a_hi = abufs.at[k][arow, pl.ds(L, L)]
        prod = u_lo * a_lo + u_hi * a_hi
        return lax.reduce_sum(prod, (0,))

    for k in range(K):
        _issue(k, k)

    zeros = jnp.zeros((L,), jnp.float32)

    @pl.loop(0, BPW // K, init_carry=zeros)
    def _eiter(i, acc):
        e0 = i * K
        for k in range(K):
            _drain(k)
            s = _compute(e0 + k, k)
            lane = jnp.bitwise_and(e0 + k, L - 1)
            acc = jnp.where(iota == lane, s, acc)

            @pl.when(i < BPW // K - 1)
            def _():
                _issue(e0 + k + K, k)

        @pl.when(jnp.bitwise_and(i, (L // K) - 1) == (L // K) - 1)
        def _():
            ps = 1.0 / (1.0 + jnp.exp(-acc))
            g = lax.div(i, L // K)
            oid = (g * L + iota) * 2
            plsc.store_scatter(out_v, [oid], 1.0 - ps)
            plsc.store_scatter(out_v, [oid + 1], ps)

        return acc

    pltpu.sync_copy(out_v, out_hbm.at[pl.ds(base * 2, BPW * 2)])


def kernel(user_ids, ad_ids, user_table, ad_table):
    uids = user_ids.astype(jnp.int32)
    aids = ad_ids.astype(jnp.int32)
    utab = user_table.T
    mesh = plsc.VectorSubcoreMesh(core_axis_name="c", subcore_axis_name="s",
                                  num_cores=NC, num_subcores=NS)
    f = pl.kernel(
        _sc_body,
        out_type=jax.ShapeDtypeStruct((BATCH * 2,), jnp.float32),
        mesh=mesh,
        compiler_params=pltpu.CompilerParams(needs_layout_passes=False),
        scratch_types=[
            pltpu.VMEM((BPW + L,), jnp.int32),
            pltpu.VMEM((BPW + L,), jnp.int32),
            pltpu.VMEM((K, D, 128), jnp.float32),
            pltpu.VMEM((K, 8, D), jnp.float32),
            pltpu.VMEM((BPW,), jnp.float32),
            pltpu.VMEM((BPW * 2,), jnp.float32),
            pltpu.SemaphoreType.DMA((K,)),
        ],
    )
    out = f(uids, aids, utab, ad_table)
    return out.reshape(BATCH, 2)


# speedup vs baseline: 3.1085x; 1.0151x over previous
import jax
import jax.numpy as jnp
from jax import lax
from jax.experimental import pallas as pl
from jax.experimental.pallas import tpu as pltpu
from jax.experimental.pallas import tpu_sc as plsc

NC = 2
NS = 16
L = 16
NW = NC * NS
BATCH = 16384
D = 32
BPW = BATCH // NW
K = 16


def _sc_body(uids_hbm, aids_hbm, utab_hbm, atab_hbm, out_hbm,
             uids_v, aids_v, ubufs, abufs, dots_v, out_v, sems):
    wid = lax.axis_index("s") * NC + lax.axis_index("c")
    base = wid * BPW

    pltpu.sync_copy(uids_hbm.at[pl.ds(base, BPW)], uids_v.at[pl.ds(0, BPW)])
    pltpu.sync_copy(aids_hbm.at[pl.ds(base, BPW)], aids_v.at[pl.ds(0, BPW)])

    def _sread(ref, e):
        return ref[pl.ds(e, L)][0]

    iota = lax.iota(jnp.int32, L)
    lo_rows = iota
    hi_rows = iota + L

    def _issue(e, k):
        uid = _sread(uids_v, e)
        aid = _sread(aids_v, e)
        ub = pl.multiple_of(lax.shift_left(lax.shift_right_logical(uid, 7), 7),
                            128)
        ar = pl.multiple_of(lax.shift_left(lax.shift_right_logical(aid, 3), 3),
                            8)
        pltpu.async_copy(utab_hbm.at[:, pl.ds(ub, 128)], ubufs.at[k], sems.at[k])
        pltpu.async_copy(atab_hbm.at[pl.ds(ar, 8), :], abufs.at[k], sems.at[k])

    def _drain(k):
        pltpu.make_async_copy(
            utab_hbm.at[:, pl.ds(0, 128)], ubufs.at[k], sems.at[k]).wait()
        pltpu.make_async_copy(
            atab_hbm.at[pl.ds(0, 8), :], abufs.at[k], sems.at[k]).wait()

    def _compute(e, k):
        uid = _sread(uids_v, e)
        aid = _sread(aids_v, e)
        uc = jnp.full((L,), jnp.bitwise_and(uid, 127), jnp.int32)
        arow = jnp.bitwise_and(aid, 7)
        u_lo = plsc.load_gather(ubufs.at[k], [lo_rows, uc])
        u_hi = plsc.load_gather(ubufs.at[k], [hi_rows, uc])
        a_lo = abufs.at[k][arow, pl.ds(0, L)]
        a_hi = abufs.at[k][arow, pl.ds(L, L)]
        prod = u_lo * a_lo + u_hi * a_hi
        return lax.reduce_sum(prod, (0,))

    for k in range(K):
        _issue(k, k)

    zeros = jnp.zeros((L,), jnp.float32)

    @pl.loop(0, BPW // K, init_carry=zeros)
    def _eiter(i, acc):
        e0 = i * K
        for k in range(K):
            _drain(k)
            s = _compute(e0 + k, k)
            lane = jnp.bitwise_and(e0 + k, L - 1)
            acc = jnp.where(iota == lane, s, acc)

            @pl.when(i < BPW // K - 1)
            def _():
                _issue(e0 + k + K, k)

        @pl.when(jnp.bitwise_and(i, (L // K) - 1) == (L // K) - 1)
        def _():
            ps = 1.0 / (1.0 + jnp.exp(-acc))
            g = lax.div(i, L // K)
            oid = (g * L + iota) * 2
            plsc.store_scatter(out_v, [oid], 1.0 - ps)
            plsc.store_scatter(out_v, [oid + 1], ps)

        return acc

    pltpu.sync_copy(out_v, out_hbm.at[pl.ds(base * 2, BPW * 2)])


def kernel(user_ids, ad_ids, user_table, ad_table):
    uids = user_ids.astype(jnp.int32)
    aids = ad_ids.astype(jnp.int32)
    utab = user_table.T
    mesh = plsc.VectorSubcoreMesh(core_axis_name="c", subcore_axis_name="s",
                                  num_cores=NC, num_subcores=NS)
    f = pl.kernel(
        _sc_body,
        out_type=jax.ShapeDtypeStruct((BATCH * 2,), jnp.float32),
        mesh=mesh,
        compiler_params=pltpu.CompilerParams(needs_layout_passes=False),
        scratch_types=[
            pltpu.VMEM((BPW + L,), jnp.int32),
            pltpu.VMEM((BPW + L,), jnp.int32),
            pltpu.VMEM((K, D, 128), jnp.float32),
            pltpu.VMEM((K, 8, D), jnp.float32),
            pltpu.VMEM((BPW,), jnp.float32),
            pltpu.VMEM((BPW * 2,), jnp.float32),
            pltpu.SemaphoreType.DMA((K,)),
        ],
    )
    out = f(uids, aids, utab, ad_table)
    return out.reshape(BATCH, 2)
